# baseline (speedup 1.0000x reference)
import functools

import jax
import jax.numpy as jnp
from jax import lax
from jax.experimental import pallas as pl
from jax.experimental.pallas import tpu as pltpu
from jax.experimental.pallas import tpu_sc as plsc

N_NODES = 10000
N_EDGES = 320000
C = 128

NC = 2
NS = 16
NW = NC * NS
K = 128
G = 2
NCH = 80
NGRP = NCH // G
EPW = K * NCH
EPAD = EPW * NW
NPAD = 10240
RPT = NPAD // NS

_mesh = plsc.VectorSubcoreMesh(
    core_axis_name="c", subcore_axis_name="s", num_cores=NC, num_subcores=NS)


@functools.partial(
    pl.kernel,
    out_type=jax.ShapeDtypeStruct((NC, NPAD), jnp.float32),
    mesh=_mesh,
    scratch_types=[
        pltpu.VMEM((NCH, K), jnp.int32),
        pltpu.VMEM((K,), jnp.float32),
        pltpu.VMEM((RPT,), jnp.float32),
        pltpu.VMEM_SHARED((NPAD,), jnp.float32),
    ],
)
def _deg_kernel(dst_hbm, deg_out, idx_v, ones_v, zero_v, deg_sh):
  c = lax.axis_index("c")
  t = lax.axis_index("s")
  wid = c * NS + t

  pltpu.sync_copy(dst_hbm.at[wid], idx_v)

  @pl.loop(0, K // 16)
  def _(i):
    ones_v[pl.ds(i * 16, 16)] = jnp.ones((16,), jnp.float32)

  @pl.loop(0, RPT // 16)
  def _(i):
    zero_v[pl.ds(i * 16, 16)] = jnp.zeros((16,), jnp.float32)

  pltpu.sync_copy(zero_v, deg_sh.at[pl.ds(t * RPT, RPT)])
  plsc.subcore_barrier()

  @pl.loop(0, NCH)
  def _(j):
    pltpu.sync_copy(ones_v, deg_sh.at[idx_v.at[j]], add=True)

  plsc.subcore_barrier()
  pltpu.sync_copy(deg_sh.at[pl.ds(t * RPT, RPT)],
                  deg_out.at[c, pl.ds(t * RPT, RPT)])


def _prep_body(x_ref, w1_ref, degp_ref, hp_ref, dis_ref):
  deg = degp_ref[0] + degp_ref[1] + 1.0
  dis = lax.rsqrt(deg)
  h = jnp.dot(x_ref[...], w1_ref[...], preferred_element_type=jnp.float32)
  hp_ref[...] = h * dis
  dis_ref[...] = dis


def _prep(xp, W1, deg_parts):
  B = 1024
  grid = NPAD // B
  return pl.pallas_call(
      _prep_body,
      grid=(grid,),
      in_specs=[
          pl.BlockSpec((B, C), lambda i: (i, 0)),
          pl.BlockSpec((C, C), lambda i: (0, 0)),
          pl.BlockSpec((NC, B, 1), lambda i: (0, i, 0)),
      ],
      out_specs=[
          pl.BlockSpec((B, C), lambda i: (i, 0)),
          pl.BlockSpec((B, 1), lambda i: (i, 0)),
      ],
      out_shape=[
          jax.ShapeDtypeStruct((NPAD, C), jnp.float32),
          jax.ShapeDtypeStruct((NPAD, 1), jnp.float32),
      ],
  )(xp, W1, deg_parts)


@functools.partial(
    pl.kernel,
    out_type=(
        jax.ShapeDtypeStruct((NC, NPAD, C), jnp.float32),
        jax.ShapeDtypeStruct((NC, NPAD), jnp.float32),
    ),
    mesh=_mesh,
    scratch_types=[
        pltpu.VMEM((4, K), jnp.int32),
        pltpu.VMEM((NCH, K), jnp.int32),
        pltpu.VMEM((2, K), jnp.float32),
        pltpu.VMEM((2, K, C), jnp.float32),
        pltpu.VMEM((RPT,), jnp.float32),
        pltpu.VMEM_SHARED((NPAD, C), jnp.float32),
        pltpu.VMEM_SHARED((NPAD,), jnp.float32),
        pltpu.SemaphoreType.DMA,
        pltpu.SemaphoreType.DMA,
        pltpu.SemaphoreType.DMA,
    ],
)
def _edge_kernel(src_hbm, dst_hbm, hp_hbm, dis_hbm,
                 acc_out, s_out,
                 srcb_v, dst_v, sval_v, rows_v, zero_v,
                 acc_sh, s_sh, sem_g, sem_i, sem_s):
  c = lax.axis_index("c")
  t = lax.axis_index("s")
  wid = c * NS + t

  pltpu.sync_copy(dst_hbm.at[wid], dst_v)

  @pl.loop(0, K)
  def _(i):
    for j in range(C // 16):
      rows_v[0, i, pl.ds(j * 16, 16)] = jnp.zeros((16,), jnp.float32)

  @pl.loop(0, RPT // 16)
  def _(i):
    zero_v[pl.ds(i * 16, 16)] = jnp.zeros((16,), jnp.float32)

  @pl.loop(0, RPT // K)
  def _(i):
    pltpu.sync_copy(rows_v.at[0], acc_sh.at[pl.ds(t * RPT + i * K, K)])

  pltpu.sync_copy(zero_v, s_sh.at[pl.ds(t * RPT, RPT)])
  pltpu.sync_copy(src_hbm.at[wid, 0], srcb_v.at[0])

  plsc.subcore_barrier()

  def _gather(j, b, sb):
    pltpu.async_copy(hp_hbm.at[srcb_v.at[sb]], rows_v.at[b], sem_g)
    pltpu.async_copy(dis_hbm.at[dst_v.at[j]], sval_v.at[b], sem_g)

  def _wait_gather(b):
    pltpu.make_async_copy(hp_hbm.at[srcb_v.at[0]], rows_v.at[b], sem_g).wait()
    pltpu.make_async_copy(dis_hbm.at[dst_v.at[0]], sval_v.at[b], sem_g).wait()

  def _scatter(j, b, sb):
    pltpu.async_copy(rows_v.at[b], acc_sh.at[dst_v.at[j]], sem_s, add=True)
    pltpu.async_copy(sval_v.at[b], s_sh.at[srcb_v.at[sb]], sem_s, add=True)

  def _wait_scatter(b):
    pltpu.make_async_copy(rows_v.at[b], acc_sh.at[dst_v.at[0]], sem_s).wait()
    pltpu.make_async_copy(sval_v.at[b], s_sh.at[srcb_v.at[0]], sem_s).wait()

  def _step(jj, p):
    j = jj + p
    b = p % 2
    _wait_gather(b)

    if p > 0:
      _wait_scatter(1 - b)
    else:
      @pl.when(jj > 0)
      def _():
        _wait_scatter(1 - b)

    @pl.when(j + 1 < NCH)
    def _():
      pltpu.make_async_copy(src_hbm.at[wid, 0],
                            srcb_v.at[(p + 1) % 4], sem_i).wait()
      _gather(j + 1, 1 - b, (p + 1) % 4)

    _scatter(j, b, p % 4)

    @pl.when(j + 2 < NCH)
    def _():
      pltpu.async_copy(src_hbm.at[wid, j + 2], srcb_v.at[(p + 2) % 4], sem_i)

  _gather(0, 0, 0)
  pltpu.async_copy(src_hbm.at[wid, 1], srcb_v.at[1], sem_i)

  @pl.loop(0, NCH, step=4)
  def _(jj):
    for p in range(4):
      _step(jj, p)

  _wait_scatter(1)

  plsc.subcore_barrier()
  pltpu.sync_copy(acc_sh.at[pl.ds(t * RPT, RPT)],
                  acc_out.at[c, pl.ds(t * RPT, RPT)])
  pltpu.sync_copy(s_sh.at[pl.ds(t * RPT, RPT)],
                  s_out.at[c, pl.ds(t * RPT, RPT)])


def _final_body(hp_ref, dis_ref, acc_ref, s_ref, b1_ref, w2_ref, b2_ref,
                out_ref, pool_ref):
  b = pl.program_id(0)
  nb = pl.num_programs(0)
  B = hp_ref.shape[0]

  dis = dis_ref[...]
  a = acc_ref[0] + acc_ref[1] + hp_ref[...]
  h1 = dis * a + b1_ref[...]
  r = jnp.maximum(h1, 0.0)
  w = dis * (s_ref[0] + s_ref[1] + dis)
  node = b * B + lax.broadcasted_iota(jnp.int32, (B, 1), 0)
  w = jnp.where(node < N_NODES, w, 0.0)
  part = jnp.sum(w * r, axis=0, keepdims=True)

  @pl.when(b == 0)
  def _():
    pool_ref[...] = jnp.zeros_like(pool_ref)

  pool_ref[...] += part

  @pl.when(b == nb - 1)
  def _():
    pooled = pool_ref[...] * (1.0 / N_NODES)
    out_ref[...] = jnp.dot(pooled, w2_ref[...],
                           preferred_element_type=jnp.float32) + b2_ref[...]


def _final(hp, dis, acc_parts, s_parts, b1, W2, b2):
  B = 1024
  grid = NPAD // B
  return pl.pallas_call(
      _final_body,
      grid=(grid,),
      in_specs=[
          pl.BlockSpec((B, C), lambda i: (i, 0)),
          pl.BlockSpec((B, 1), lambda i: (i, 0)),
          pl.BlockSpec((NC, B, C), lambda i: (0, i, 0)),
          pl.BlockSpec((NC, B, 1), lambda i: (0, i, 0)),
          pl.BlockSpec((1, C), lambda i: (0, 0)),
          pl.BlockSpec((C, C), lambda i: (0, 0)),
          pl.BlockSpec((1, C), lambda i: (0, 0)),
      ],
      out_specs=pl.BlockSpec((1, C), lambda i: (0, 0)),
      out_shape=jax.ShapeDtypeStruct((1, C), jnp.float32),
      scratch_shapes=[pltpu.VMEM((1, C), jnp.float32)],
  )(hp, dis, acc_parts, s_parts, b1, W2, b2)


def kernel(x, edge_index, W1, b1, W2, b2):
  src = edge_index[0].astype(jnp.int32)
  dst = edge_index[1].astype(jnp.int32)

  npadded = EPAD - N_EDGES
  pad_idx = N_NODES + (jnp.arange(npadded, dtype=jnp.int32) % (NPAD - N_NODES))
  srcp = jnp.concatenate([src, pad_idx]).reshape(NW, NCH, K)
  dstp = jnp.concatenate([dst, pad_idx]).reshape(NW, NCH, K)

  xp = jnp.pad(x, ((0, NPAD - N_NODES), (0, 0)))

  deg_parts = _deg_kernel(dstp)
  hp, dis = _prep(xp, W1, deg_parts.reshape(NC, NPAD, 1))
  acc_parts, s_parts = _edge_kernel(srcp, dstp, hp, dis.reshape(NPAD))
  return _final(hp, dis, acc_parts, s_parts.reshape(NC, NPAD, 1),
                b1.reshape(1, C), W2, b2.reshape(1, C))


# speedup vs baseline: 1.0209x; 1.0209x over previous
import functools

import jax
import jax.numpy as jnp
from jax import lax
from jax.experimental import pallas as pl
from jax.experimental.pallas import tpu as pltpu
from jax.experimental.pallas import tpu_sc as plsc

N_NODES = 10000
N_EDGES = 320000
C = 128

NC = 2
NS = 16
NW = NC * NS
K = 128
G = 2
NCH = 80
NGRP = NCH // G
EPW = K * NCH
EPAD = EPW * NW
NPAD = 10240
RPT = NPAD // NS

_mesh = plsc.VectorSubcoreMesh(
    core_axis_name="c", subcore_axis_name="s", num_cores=NC, num_subcores=NS)


@functools.partial(
    pl.kernel,
    out_type=jax.ShapeDtypeStruct((NC, NPAD), jnp.float32),
    mesh=_mesh,
    scratch_types=[
        pltpu.VMEM((NCH, K), jnp.int32),
        pltpu.VMEM((K,), jnp.float32),
        pltpu.VMEM((RPT,), jnp.float32),
        pltpu.VMEM_SHARED((NPAD,), jnp.float32),
        pltpu.SemaphoreType.DMA,
    ],
)
def _deg_kernel(dst_hbm, deg_out, idx_v, ones_v, zero_v, deg_sh, sem_d):
  c = lax.axis_index("c")
  t = lax.axis_index("s")
  wid = c * NS + t

  pltpu.sync_copy(dst_hbm.at[wid], idx_v)

  @pl.loop(0, K // 16)
  def _(i):
    ones_v[pl.ds(i * 16, 16)] = jnp.ones((16,), jnp.float32)

  @pl.loop(0, RPT // 16)
  def _(i):
    zero_v[pl.ds(i * 16, 16)] = jnp.zeros((16,), jnp.float32)

  pltpu.sync_copy(zero_v, deg_sh.at[pl.ds(t * RPT, RPT)])
  plsc.subcore_barrier()

  @pl.loop(0, NCH)
  def _(j):
    pltpu.async_copy(ones_v, deg_sh.at[idx_v.at[j]], sem_d, add=True)

  @pl.loop(0, NCH)
  def _(j):
    pltpu.make_async_copy(ones_v, deg_sh.at[idx_v.at[0]], sem_d).wait()

  plsc.subcore_barrier()
  pltpu.sync_copy(deg_sh.at[pl.ds(t * RPT, RPT)],
                  deg_out.at[c, pl.ds(t * RPT, RPT)])


def _prep_body(x_ref, w1_ref, degp_ref, hp_ref, dis_ref):
  deg = degp_ref[0] + degp_ref[1] + 1.0
  dis = lax.rsqrt(deg)
  h = jnp.dot(x_ref[...], w1_ref[...], preferred_element_type=jnp.float32)
  hp_ref[...] = h * dis
  dis_ref[...] = dis


def _prep(xp, W1, deg_parts):
  B = 1024
  grid = NPAD // B
  return pl.pallas_call(
      _prep_body,
      grid=(grid,),
      in_specs=[
          pl.BlockSpec((B, C), lambda i: (i, 0)),
          pl.BlockSpec((C, C), lambda i: (0, 0)),
          pl.BlockSpec((NC, B, 1), lambda i: (0, i, 0)),
      ],
      out_specs=[
          pl.BlockSpec((B, C), lambda i: (i, 0)),
          pl.BlockSpec((B, 1), lambda i: (i, 0)),
      ],
      out_shape=[
          jax.ShapeDtypeStruct((NPAD, C), jnp.float32),
          jax.ShapeDtypeStruct((NPAD, 1), jnp.float32),
      ],
  )(xp, W1, deg_parts)


@functools.partial(
    pl.kernel,
    out_type=(
        jax.ShapeDtypeStruct((NC, NPAD, C), jnp.float32),
        jax.ShapeDtypeStruct((NC, NPAD), jnp.float32),
    ),
    mesh=_mesh,
    scratch_types=[
        pltpu.VMEM((4, K), jnp.int32),
        pltpu.VMEM((NCH, K), jnp.int32),
        pltpu.VMEM((2, K), jnp.float32),
        pltpu.VMEM((2, K, C), jnp.float32),
        pltpu.VMEM((RPT,), jnp.float32),
        pltpu.VMEM_SHARED((NPAD, C), jnp.float32),
        pltpu.VMEM_SHARED((NPAD,), jnp.float32),
        pltpu.SemaphoreType.DMA,
        pltpu.SemaphoreType.DMA,
        pltpu.SemaphoreType.DMA,
    ],
)
def _edge_kernel(src_hbm, dst_hbm, hp_hbm, dis_hbm,
                 acc_out, s_out,
                 srcb_v, dst_v, sval_v, rows_v, zero_v,
                 acc_sh, s_sh, sem_g, sem_i, sem_s):
  c = lax.axis_index("c")
  t = lax.axis_index("s")
  wid = c * NS + t

  pltpu.sync_copy(dst_hbm.at[wid], dst_v)

  @pl.loop(0, K)
  def _(i):
    for j in range(C // 16):
      rows_v[0, i, pl.ds(j * 16, 16)] = jnp.zeros((16,), jnp.float32)

  @pl.loop(0, RPT // 16)
  def _(i):
    zero_v[pl.ds(i * 16, 16)] = jnp.zeros((16,), jnp.float32)

  @pl.loop(0, RPT // K)
  def _(i):
    pltpu.sync_copy(rows_v.at[0], acc_sh.at[pl.ds(t * RPT + i * K, K)])

  pltpu.sync_copy(zero_v, s_sh.at[pl.ds(t * RPT, RPT)])
  pltpu.sync_copy(src_hbm.at[wid, 0], srcb_v.at[0])

  plsc.subcore_barrier()

  def _gather(j, b, sb):
    pltpu.async_copy(hp_hbm.at[srcb_v.at[sb]], rows_v.at[b], sem_g)
    pltpu.async_copy(dis_hbm.at[dst_v.at[j]], sval_v.at[b], sem_g)

  def _wait_gather(b):
    pltpu.make_async_copy(hp_hbm.at[srcb_v.at[0]], rows_v.at[b], sem_g).wait()
    pltpu.make_async_copy(dis_hbm.at[dst_v.at[0]], sval_v.at[b], sem_g).wait()

  def _scatter(j, b, sb):
    pltpu.async_copy(rows_v.at[b], acc_sh.at[dst_v.at[j]], sem_s, add=True)
    pltpu.async_copy(sval_v.at[b], s_sh.at[srcb_v.at[sb]], sem_s, add=True)

  def _wait_scatter(b):
    pltpu.make_async_copy(rows_v.at[b], acc_sh.at[dst_v.at[0]], sem_s).wait()
    pltpu.make_async_copy(sval_v.at[b], s_sh.at[srcb_v.at[0]], sem_s).wait()

  def _step(jj, p):
    j = jj + p
    b = p % 2
    _wait_gather(b)

    if p > 0:
      _wait_scatter(1 - b)
    else:
      @pl.when(jj > 0)
      def _():
        _wait_scatter(1 - b)

    @pl.when(j + 1 < NCH)
    def _():
      pltpu.make_async_copy(src_hbm.at[wid, 0],
                            srcb_v.at[(p + 1) % 4], sem_i).wait()
      _gather(j + 1, 1 - b, (p + 1) % 4)

    _scatter(j, b, p % 4)

    @pl.when(j + 2 < NCH)
    def _():
      pltpu.async_copy(src_hbm.at[wid, j + 2], srcb_v.at[(p + 2) % 4], sem_i)

  _gather(0, 0, 0)
  pltpu.async_copy(src_hbm.at[wid, 1], srcb_v.at[1], sem_i)

  @pl.loop(0, NCH, step=4)
  def _(jj):
    for p in range(4):
      _step(jj, p)

  _wait_scatter(1)

  plsc.subcore_barrier()
  pltpu.sync_copy(acc_sh.at[pl.ds(t * RPT, RPT)],
                  acc_out.at[c, pl.ds(t * RPT, RPT)])
  pltpu.sync_copy(s_sh.at[pl.ds(t * RPT, RPT)],
                  s_out.at[c, pl.ds(t * RPT, RPT)])


def _final_body(hp_ref, dis_ref, acc_ref, s_ref, b1_ref, w2_ref, b2_ref,
                out_ref, pool_ref):
  b = pl.program_id(0)
  nb = pl.num_programs(0)
  B = hp_ref.shape[0]

  dis = dis_ref[...]
  a = acc_ref[0] + acc_ref[1] + hp_ref[...]
  h1 = dis * a + b1_ref[...]
  r = jnp.maximum(h1, 0.0)
  w = dis * (s_ref[0] + s_ref[1] + dis)
  node = b * B + lax.broadcasted_iota(jnp.int32, (B, 1), 0)
  w = jnp.where(node < N_NODES, w, 0.0)
  part = jnp.sum(w * r, axis=0, keepdims=True)

  @pl.when(b == 0)
  def _():
    pool_ref[...] = jnp.zeros_like(pool_ref)

  pool_ref[...] += part

  @pl.when(b == nb - 1)
  def _():
    pooled = pool_ref[...] * (1.0 / N_NODES)
    out_ref[...] = jnp.dot(pooled, w2_ref[...],
                           preferred_element_type=jnp.float32) + b2_ref[...]


def _final(hp, dis, acc_parts, s_parts, b1, W2, b2):
  B = 1024
  grid = NPAD // B
  return pl.pallas_call(
      _final_body,
      grid=(grid,),
      in_specs=[
          pl.BlockSpec((B, C), lambda i: (i, 0)),
          pl.BlockSpec((B, 1), lambda i: (i, 0)),
          pl.BlockSpec((NC, B, C), lambda i: (0, i, 0)),
          pl.BlockSpec((NC, B, 1), lambda i: (0, i, 0)),
          pl.BlockSpec((1, C), lambda i: (0, 0)),
          pl.BlockSpec((C, C), lambda i: (0, 0)),
          pl.BlockSpec((1, C), lambda i: (0, 0)),
      ],
      out_specs=pl.BlockSpec((1, C), lambda i: (0, 0)),
      out_shape=jax.ShapeDtypeStruct((1, C), jnp.float32),
      scratch_shapes=[pltpu.VMEM((1, C), jnp.float32)],
  )(hp, dis, acc_parts, s_parts, b1, W2, b2)


def kernel(x, edge_index, W1, b1, W2, b2):
  src = edge_index[0].astype(jnp.int32)
  dst = edge_index[1].astype(jnp.int32)

  npadded = EPAD - N_EDGES
  pad_idx = N_NODES + (jnp.arange(npadded, dtype=jnp.int32) % (NPAD - N_NODES))
  srcp = jnp.concatenate([src, pad_idx]).reshape(NW, NCH, K)
  dstp = jnp.concatenate([dst, pad_idx]).reshape(NW, NCH, K)

  xp = jnp.pad(x, ((0, NPAD - N_NODES), (0, 0)))

  deg_parts = _deg_kernel(dstp)
  hp, dis = _prep(xp, W1, deg_parts.reshape(NC, NPAD, 1))
  acc_parts, s_parts = _edge_kernel(srcp, dstp, hp, dis.reshape(NPAD))
  return _final(hp, dis, acc_parts, s_parts.reshape(NC, NPAD, 1),
                b1.reshape(1, C), W2, b2.reshape(1, C))
